# baseline (speedup 1.0000x reference)
import functools

import jax
import jax.numpy as jnp
from jax import lax
from jax.experimental import pallas as pl
from jax.experimental.pallas import tpu as pltpu
from jax.experimental.pallas import tpu_sc as plsc

N_NODES = 10000
N_EDGES = 320000
D_FEAT = 128
EDGE_DIM = 16

NC = 2
NS = 16
NW = NC * NS
CHUNK = 640
SUB = CHUNK // 128
N_CHUNKS = N_EDGES // CHUNK
N_TCOLS = N_EDGES // 128
MAX_WCHUNKS = N_CHUNKS // NW + 1
NBUF = 2

ROWS_BLK = N_NODES


def _tables_body(x_ref, wb_ref, bb_ref, wr_ref, ws_ref, br_ref, ps_b_ref,
                 pr_ref, ps_ref):
    t = jnp.tanh(
        jnp.dot(x_ref[...], wb_ref[...], preferred_element_type=jnp.float32)
        + bb_ref[...]
    )
    t_r = t.reshape(ROWS_BLK // 8, 8 * D_FEAT)
    pr_ref[...] = (jnp.dot(t_r, wr_ref[...], preferred_element_type=jnp.float32)
                   + br_ref[...])
    ps_ref[...] = (jnp.dot(t_r, ws_ref[...], preferred_element_type=jnp.float32)
                   + ps_b_ref[...])


def _node_tables(x, W_base, b_base, W_edge, b_edge):
    eye8 = jnp.eye(8, dtype=jnp.float32)
    w_r = jnp.kron(eye8, 2.0 * W_edge[:D_FEAT])
    w_s = jnp.kron(eye8, 2.0 * W_edge[D_FEAT:])
    b_r = jnp.tile(2.0 * b_edge, 8).reshape(1, 128)
    b_s = jnp.zeros((1, 128), jnp.float32)
    grid = (N_NODES // ROWS_BLK,)
    return pl.pallas_call(
        _tables_body,
        grid=grid,
        in_specs=[
            pl.BlockSpec((ROWS_BLK, D_FEAT), lambda i: (i, 0)),
            pl.BlockSpec((D_FEAT, D_FEAT), lambda i: (0, 0)),
            pl.BlockSpec((1, D_FEAT), lambda i: (0, 0)),
            pl.BlockSpec((8 * D_FEAT, 128), lambda i: (0, 0)),
            pl.BlockSpec((8 * D_FEAT, 128), lambda i: (0, 0)),
            pl.BlockSpec((1, 128), lambda i: (0, 0)),
            pl.BlockSpec((1, 128), lambda i: (0, 0)),
        ],
        out_specs=[
            pl.BlockSpec((ROWS_BLK // 8, 128), lambda i: (i, 0)),
            pl.BlockSpec((ROWS_BLK // 8, 128), lambda i: (i, 0)),
        ],
        out_shape=[
            jax.ShapeDtypeStruct((N_NODES // 8, 128), jnp.float32),
            jax.ShapeDtypeStruct((N_NODES // 8, 128), jnp.float32),
        ],
    )(x, W_base, b_base.reshape(1, D_FEAT), w_r, w_s, b_r, b_s)


def _edge_body(pr_hbm, ps_hbm, ridx_hbm, sidx_hbm, out_hbm,
               ridx_v, sidx_v,
               rbuf0, rbuf1, sbuf0, sbuf1, obuf0, obuf1,
               pr_s, ps_s,
               sem_i0, sem_i1, sem_o0, sem_o1, sem_t):
    rbufs, sbufs = [rbuf0, rbuf1], [sbuf0, sbuf1]
    obufs = [obuf0, obuf1]
    sem_is, sem_os = [sem_i0, sem_i1], [sem_o0, sem_o1]
    sid = lax.axis_index("s")
    wid = sid * NC + lax.axis_index("c")

    @pl.when(sid == 0)
    def _():
        pltpu.async_copy(pr_hbm, pr_s, sem_t)
        pltpu.async_copy(ps_hbm, ps_s, sem_t)
    lo_c = lax.shift_right_logical(N_CHUNKS * wid, 5)
    hi_c = lax.shift_right_logical(N_CHUNKS * (wid + 1), 5)
    n_c = hi_c - lo_c
    e_lo = lo_c * CHUNK
    pltpu.sync_copy(ridx_hbm.at[pl.ds(e_lo, MAX_WCHUNKS * CHUNK)], ridx_v)
    pltpu.sync_copy(sidx_hbm.at[pl.ds(e_lo, MAX_WCHUNKS * CHUNK)], sidx_v)
    lanes = jnp.arange(EDGE_DIM, dtype=jnp.int32)

    @pl.when(sid == 0)
    def _():
        pltpu.make_async_copy(pr_hbm, pr_s, sem_t).wait()
        pltpu.make_async_copy(ps_hbm, ps_s, sem_t).wait()

    plsc.subcore_barrier()

    def start_in(k, b):
        idx_r = ridx_v.at[pl.ds(k * CHUNK, CHUNK)]
        idx_s = sidx_v.at[pl.ds(k * CHUNK, CHUNK)]
        pltpu.async_copy(pr_s.at[idx_r], rbufs[b], sem_is[b])
        pltpu.async_copy(ps_s.at[idx_s], sbufs[b], sem_is[b])

    def wait_in(k, b):
        idx_r = ridx_v.at[pl.ds(k * CHUNK, CHUNK)]
        pltpu.make_async_copy(pr_s.at[idx_r], rbufs[b], sem_is[b]).wait()
        pltpu.make_async_copy(pr_s.at[idx_r], sbufs[b], sem_is[b]).wait()

    def start_out(k, b):
        c0 = (lo_c + k) * SUB
        for fr in range(2):
            for sub in range(SUB):
                pltpu.async_copy(
                    obufs[b].at[pl.ds(fr * 8, 8), pl.ds(sub * 128, 128)],
                    out_hbm.at[fr, c0 + sub], sem_os[b])

    def wait_out(k, b):
        c0 = (lo_c + k) * SUB
        for fr in range(2):
            for sub in range(SUB):
                pltpu.make_async_copy(
                    obufs[b].at[pl.ds(fr * 8, 8), pl.ds(sub * 128, 128)],
                    out_hbm.at[fr, c0 + sub], sem_os[b]).wait()

    for b in range(NBUF):
        start_in(b, b)

    @pl.loop(0, MAX_WCHUNKS + (-MAX_WCHUNKS) % NBUF, step=NBUF)
    def outer(k0):
        for b in range(NBUF):
            k = k0 + b

            @pl.when(k < n_c)
            def _():
                wait_in(k, b)

                @pl.when(k >= NBUF)
                def _():
                    wait_out(k - NBUF, b)

                rb, sb, ob = rbufs[b], sbufs[b], obufs[b]

                @plsc.parallel_loop(0, CHUNK, unroll=16)
                def rows(i):
                    e = jnp.exp(rb[i] + sb[i])
                    v = 1.0 - 2.0 / (e + 1.0)
                    plsc.store_scatter(
                        ob, [lanes, jnp.full((EDGE_DIM,), i, jnp.int32)], v)

                start_out(k, b)

                @pl.when(k + NBUF < n_c)
                def _():
                    start_in(k + NBUF, b)

    for b in range(NBUF):
        k_last = n_c - 1 - ((n_c - 1 - b) % NBUF)
        wait_out(k_last, b)


def _edge_update(pr, ps, ridx, sidx):
    mesh = plsc.VectorSubcoreMesh(core_axis_name="c", subcore_axis_name="s")
    f = pl.kernel(
        _edge_body,
        out_type=jax.ShapeDtypeStruct((2, N_TCOLS, 8, 128), jnp.float32),
        mesh=mesh,
        scratch_types=[
            pltpu.VMEM((MAX_WCHUNKS * CHUNK,), jnp.int32),
            pltpu.VMEM((MAX_WCHUNKS * CHUNK,), jnp.int32),
        ] + [pltpu.VMEM((CHUNK, EDGE_DIM), jnp.float32)] * (2 * NBUF)
          + [pltpu.VMEM((EDGE_DIM, CHUNK + 1), jnp.float32)] * NBUF
          + [pltpu.VMEM_SHARED((N_NODES, EDGE_DIM), jnp.float32)] * 2
          + [pltpu.SemaphoreType.DMA] * (2 * NBUF + 1),
        compiler_params=pltpu.CompilerParams(use_tc_tiling_on_sc=False,
                                             needs_layout_passes=False),
    )
    return f(pr, ps, ridx, sidx)


def kernel(x, senders, receivers, W_base, b_base, W_edge, b_edge):
    pr_c, ps_c = _node_tables(x, W_base, b_base, W_edge, b_edge)
    pr = pr_c.reshape(N_NODES, EDGE_DIM)
    ps = ps_c.reshape(N_NODES, EDGE_DIM)
    out4 = _edge_update(pr, ps, receivers, senders)
    return out4.transpose(1, 3, 0, 2).reshape(N_EDGES, EDGE_DIM)


# speedup vs baseline: 1.0202x; 1.0202x over previous
import functools

import jax
import jax.numpy as jnp
from jax import lax
from jax.experimental import pallas as pl
from jax.experimental.pallas import tpu as pltpu
from jax.experimental.pallas import tpu_sc as plsc

N_NODES = 10000
N_EDGES = 320000
D_FEAT = 128
EDGE_DIM = 16

NC = 2
NS = 16
NW = NC * NS
CHUNK = 512
SUB = CHUNK // 128
N_CHUNKS = N_EDGES // CHUNK
N_TCOLS = N_EDGES // 128
MAX_WCHUNKS = N_CHUNKS // NW + 1
NBUF = 3

ROWS_BLK = N_NODES


def _tables_body(x_ref, wb_ref, bb_ref, wr_ref, ws_ref, br_ref, ps_b_ref,
                 pr_ref, ps_ref):
    t = jnp.tanh(
        jnp.dot(x_ref[...], wb_ref[...], preferred_element_type=jnp.float32)
        + bb_ref[...]
    )
    t_r = t.reshape(ROWS_BLK // 8, 8 * D_FEAT)
    pr_ref[...] = (jnp.dot(t_r, wr_ref[...], preferred_element_type=jnp.float32)
                   + br_ref[...])
    ps_ref[...] = (jnp.dot(t_r, ws_ref[...], preferred_element_type=jnp.float32)
                   + ps_b_ref[...])


def _node_tables(x, W_base, b_base, W_edge, b_edge):
    eye8 = jnp.eye(8, dtype=jnp.float32)
    w_r = jnp.kron(eye8, 2.0 * W_edge[:D_FEAT])
    w_s = jnp.kron(eye8, 2.0 * W_edge[D_FEAT:])
    b_r = jnp.tile(2.0 * b_edge, 8).reshape(1, 128)
    b_s = jnp.zeros((1, 128), jnp.float32)
    grid = (N_NODES // ROWS_BLK,)
    return pl.pallas_call(
        _tables_body,
        grid=grid,
        in_specs=[
            pl.BlockSpec((ROWS_BLK, D_FEAT), lambda i: (i, 0)),
            pl.BlockSpec((D_FEAT, D_FEAT), lambda i: (0, 0)),
            pl.BlockSpec((1, D_FEAT), lambda i: (0, 0)),
            pl.BlockSpec((8 * D_FEAT, 128), lambda i: (0, 0)),
            pl.BlockSpec((8 * D_FEAT, 128), lambda i: (0, 0)),
            pl.BlockSpec((1, 128), lambda i: (0, 0)),
            pl.BlockSpec((1, 128), lambda i: (0, 0)),
        ],
        out_specs=[
            pl.BlockSpec((ROWS_BLK // 8, 128), lambda i: (i, 0)),
            pl.BlockSpec((ROWS_BLK // 8, 128), lambda i: (i, 0)),
        ],
        out_shape=[
            jax.ShapeDtypeStruct((N_NODES // 8, 128), jnp.float32),
            jax.ShapeDtypeStruct((N_NODES // 8, 128), jnp.float32),
        ],
    )(x, W_base, b_base.reshape(1, D_FEAT), w_r, w_s, b_r, b_s)


def _edge_body(pr_hbm, ps_hbm, ridx_hbm, sidx_hbm, out_hbm,
               ridx_v, sidx_v,
               rbuf0, rbuf1, rbuf2, sbuf0, sbuf1, sbuf2,
               obuf0, obuf1, obuf2,
               pr_s, ps_s,
               sem_i0, sem_i1, sem_i2, sem_o0, sem_o1, sem_o2, sem_t):
    rbufs, sbufs = [rbuf0, rbuf1, rbuf2], [sbuf0, sbuf1, sbuf2]
    obufs = [obuf0, obuf1, obuf2]
    sem_is, sem_os = [sem_i0, sem_i1, sem_i2], [sem_o0, sem_o1, sem_o2]
    sid = lax.axis_index("s")
    wid = sid * NC + lax.axis_index("c")

    @pl.when(sid == 0)
    def _():
        pltpu.async_copy(pr_hbm, pr_s, sem_t)
        pltpu.async_copy(ps_hbm, ps_s, sem_t)
    lo_c = lax.shift_right_logical(N_CHUNKS * wid, 5)
    hi_c = lax.shift_right_logical(N_CHUNKS * (wid + 1), 5)
    n_c = hi_c - lo_c
    e_lo = lo_c * CHUNK
    pltpu.sync_copy(ridx_hbm.at[pl.ds(e_lo, MAX_WCHUNKS * CHUNK)], ridx_v)
    pltpu.sync_copy(sidx_hbm.at[pl.ds(e_lo, MAX_WCHUNKS * CHUNK)], sidx_v)
    lanes = jnp.arange(EDGE_DIM, dtype=jnp.int32)

    @pl.when(sid == 0)
    def _():
        pltpu.make_async_copy(pr_hbm, pr_s, sem_t).wait()
        pltpu.make_async_copy(ps_hbm, ps_s, sem_t).wait()

    plsc.subcore_barrier()

    def start_in(k, b):
        idx_r = ridx_v.at[pl.ds(k * CHUNK, CHUNK)]
        idx_s = sidx_v.at[pl.ds(k * CHUNK, CHUNK)]
        pltpu.async_copy(pr_s.at[idx_r], rbufs[b], sem_is[b])
        pltpu.async_copy(ps_s.at[idx_s], sbufs[b], sem_is[b])

    def wait_in(k, b):
        idx_r = ridx_v.at[pl.ds(k * CHUNK, CHUNK)]
        pltpu.make_async_copy(pr_s.at[idx_r], rbufs[b], sem_is[b]).wait()
        pltpu.make_async_copy(pr_s.at[idx_r], sbufs[b], sem_is[b]).wait()

    def start_out(k, b):
        c0 = (lo_c + k) * SUB
        for fr in range(2):
            for sub in range(SUB):
                pltpu.async_copy(
                    obufs[b].at[pl.ds(fr * 8, 8), pl.ds(sub * 128, 128)],
                    out_hbm.at[fr, c0 + sub], sem_os[b])

    def wait_out(k, b):
        c0 = (lo_c + k) * SUB
        for fr in range(2):
            for sub in range(SUB):
                pltpu.make_async_copy(
                    obufs[b].at[pl.ds(fr * 8, 8), pl.ds(sub * 128, 128)],
                    out_hbm.at[fr, c0 + sub], sem_os[b]).wait()

    for b in range(NBUF):
        start_in(b, b)

    @pl.loop(0, MAX_WCHUNKS + (-MAX_WCHUNKS) % NBUF, step=NBUF)
    def outer(k0):
        for b in range(NBUF):
            k = k0 + b

            @pl.when(k < n_c)
            def _():
                wait_in(k, b)

                @pl.when(k >= NBUF)
                def _():
                    wait_out(k - NBUF, b)

                rb, sb, ob = rbufs[b], sbufs[b], obufs[b]

                @plsc.parallel_loop(0, CHUNK, unroll=8)
                def rows(i):
                    e = jnp.exp(rb[i] + sb[i])
                    v = 1.0 - 2.0 / (e + 1.0)
                    plsc.store_scatter(
                        ob, [lanes, jnp.full((EDGE_DIM,), i, jnp.int32)], v)

                start_out(k, b)

                @pl.when(k + NBUF < n_c)
                def _():
                    start_in(k + NBUF, b)

    for b in range(NBUF):
        k_last = n_c - 1 - ((n_c - 1 - b) % NBUF)
        wait_out(k_last, b)


def _edge_update(pr, ps, ridx, sidx):
    mesh = plsc.VectorSubcoreMesh(core_axis_name="c", subcore_axis_name="s")
    f = pl.kernel(
        _edge_body,
        out_type=jax.ShapeDtypeStruct((2, N_TCOLS, 8, 128), jnp.float32),
        mesh=mesh,
        scratch_types=[
            pltpu.VMEM((MAX_WCHUNKS * CHUNK,), jnp.int32),
            pltpu.VMEM((MAX_WCHUNKS * CHUNK,), jnp.int32),
        ] + [pltpu.VMEM((CHUNK, EDGE_DIM), jnp.float32)] * (2 * NBUF)
          + [pltpu.VMEM((EDGE_DIM, CHUNK + 1), jnp.float32)] * NBUF
          + [pltpu.VMEM_SHARED((N_NODES, EDGE_DIM), jnp.float32)] * 2
          + [pltpu.SemaphoreType.DMA] * (2 * NBUF + 1),
        compiler_params=pltpu.CompilerParams(use_tc_tiling_on_sc=False,
                                             needs_layout_passes=False),
    )
    return f(pr, ps, ridx, sidx)


def kernel(x, senders, receivers, W_base, b_base, W_edge, b_edge):
    pr_c, ps_c = _node_tables(x, W_base, b_base, W_edge, b_edge)
    pr = pr_c.reshape(N_NODES, EDGE_DIM)
    ps = ps_c.reshape(N_NODES, EDGE_DIM)
    out4 = _edge_update(pr, ps, receivers, senders)
    return out4.transpose(1, 3, 0, 2).reshape(N_EDGES, EDGE_DIM)


# speedup vs baseline: 1.0266x; 1.0063x over previous
import jax
import jax.numpy as jnp
from jax import lax
from jax.experimental import pallas as pl
from jax.experimental.pallas import tpu as pltpu
from jax.experimental.pallas import tpu_sc as plsc

N_NODES = 10000
N_EDGES = 320000
D_FEAT = 128
EDGE_DIM = 16

NC = 2
NS = 16
NW = NC * NS
CHUNK = 512
SUB = CHUNK // 128
N_CHUNKS = N_EDGES // CHUNK
N_TCOLS = N_EDGES // 128
MAX_WCHUNKS = N_CHUNKS // NW + 1
NBUF = 2

ROWS_BLK = N_NODES


def _tables_body(x_ref, wb_ref, bb_ref, wr_ref, ws_ref, br_ref, ps_b_ref,
                 pr_ref, ps_ref):
    t = jnp.tanh(
        jnp.dot(x_ref[...], wb_ref[...], preferred_element_type=jnp.float32)
        + bb_ref[...]
    )
    t_r = t.reshape(ROWS_BLK // 8, 8 * D_FEAT)
    pr_ref[...] = (jnp.dot(t_r, wr_ref[...], preferred_element_type=jnp.float32)
                   + br_ref[...])
    ps_ref[...] = (jnp.dot(t_r, ws_ref[...], preferred_element_type=jnp.float32)
                   + ps_b_ref[...])


def _node_tables(x, W_base, b_base, W_edge, b_edge):
    eye8 = jnp.eye(8, dtype=jnp.float32)
    w_r = jnp.kron(eye8, 2.0 * W_edge[:D_FEAT])
    w_s = jnp.kron(eye8, 2.0 * W_edge[D_FEAT:])
    b_r = jnp.tile(2.0 * b_edge, 8).reshape(1, 128)
    b_s = jnp.zeros((1, 128), jnp.float32)
    grid = (N_NODES // ROWS_BLK,)
    return pl.pallas_call(
        _tables_body,
        grid=grid,
        in_specs=[
            pl.BlockSpec((ROWS_BLK, D_FEAT), lambda i: (i, 0)),
            pl.BlockSpec((D_FEAT, D_FEAT), lambda i: (0, 0)),
            pl.BlockSpec((1, D_FEAT), lambda i: (0, 0)),
            pl.BlockSpec((8 * D_FEAT, 128), lambda i: (0, 0)),
            pl.BlockSpec((8 * D_FEAT, 128), lambda i: (0, 0)),
            pl.BlockSpec((1, 128), lambda i: (0, 0)),
            pl.BlockSpec((1, 128), lambda i: (0, 0)),
        ],
        out_specs=[
            pl.BlockSpec((ROWS_BLK // 8, 128), lambda i: (i, 0)),
            pl.BlockSpec((ROWS_BLK // 8, 128), lambda i: (i, 0)),
        ],
        out_shape=[
            jax.ShapeDtypeStruct((N_NODES // 8, 128), jnp.float32),
            jax.ShapeDtypeStruct((N_NODES // 8, 128), jnp.float32),
        ],
    )(x, W_base, b_base.reshape(1, D_FEAT), w_r, w_s, b_r, b_s)


def _edge_body(pr_hbm, ps_hbm, ridx_hbm, sidx_hbm, out_hbm,
               ridx_v, sidx_v,
               rbuf0, rbuf1, sbuf0, sbuf1, obuf0, obuf1,
               pr_s, ps_s,
               sem_i0, sem_i1, sem_o0, sem_o1, sem_t):
    rbufs, sbufs, obufs = [rbuf0, rbuf1], [sbuf0, sbuf1], [obuf0, obuf1]
    sem_is, sem_os = [sem_i0, sem_i1], [sem_o0, sem_o1]
    sid = lax.axis_index("s")
    wid = sid * NC + lax.axis_index("c")

    @pl.when(sid == 0)
    def _():
        pltpu.async_copy(pr_hbm, pr_s, sem_t)
        pltpu.async_copy(ps_hbm, ps_s, sem_t)
    lo_c = lax.shift_right_logical(N_CHUNKS * wid, 5)
    hi_c = lax.shift_right_logical(N_CHUNKS * (wid + 1), 5)
    n_c = hi_c - lo_c
    e_lo = lo_c * CHUNK
    pltpu.sync_copy(ridx_hbm.at[pl.ds(e_lo, MAX_WCHUNKS * CHUNK)], ridx_v)
    pltpu.sync_copy(sidx_hbm.at[pl.ds(e_lo, MAX_WCHUNKS * CHUNK)], sidx_v)
    lanes = jnp.arange(EDGE_DIM, dtype=jnp.int32)

    @pl.when(sid == 0)
    def _():
        pltpu.make_async_copy(pr_hbm, pr_s, sem_t).wait()
        pltpu.make_async_copy(ps_hbm, ps_s, sem_t).wait()

    plsc.subcore_barrier()

    def start_in(k, b):
        idx_r = ridx_v.at[pl.ds(k * CHUNK, CHUNK)]
        idx_s = sidx_v.at[pl.ds(k * CHUNK, CHUNK)]
        pltpu.async_copy(pr_s.at[idx_r], rbufs[b], sem_is[b])
        pltpu.async_copy(ps_s.at[idx_s], sbufs[b], sem_is[b])

    def wait_in(k, b):
        idx_r = ridx_v.at[pl.ds(k * CHUNK, CHUNK)]
        pltpu.make_async_copy(pr_s.at[idx_r], rbufs[b], sem_is[b]).wait()
        pltpu.make_async_copy(pr_s.at[idx_r], sbufs[b], sem_is[b]).wait()

    def start_out(k, b):
        c0 = (lo_c + k) * SUB
        for fr in range(2):
            for sub in range(SUB):
                pltpu.async_copy(
                    obufs[b].at[pl.ds(fr * 8, 8), pl.ds(sub * 128, 128)],
                    out_hbm.at[fr, c0 + sub], sem_os[b])

    def wait_out(k, b):
        c0 = (lo_c + k) * SUB
        for fr in range(2):
            for sub in range(SUB):
                pltpu.make_async_copy(
                    obufs[b].at[pl.ds(fr * 8, 8), pl.ds(sub * 128, 128)],
                    out_hbm.at[fr, c0 + sub], sem_os[b]).wait()

    for b in range(NBUF):
        start_in(b, b)

    @pl.loop(0, MAX_WCHUNKS + (-MAX_WCHUNKS) % NBUF, step=NBUF)
    def outer(k0):
        for b in range(NBUF):
            k = k0 + b

            @pl.when(k < n_c)
            def _():
                wait_in(k, b)

                @pl.when(k >= NBUF)
                def _():
                    wait_out(k - NBUF, b)

                rb, sb, ob = rbufs[b], sbufs[b], obufs[b]

                @plsc.parallel_loop(0, CHUNK, unroll=8)
                def rows(i):
                    e = jnp.exp(rb[i] + sb[i])
                    v = 1.0 - 2.0 / (e + 1.0)
                    plsc.store_scatter(
                        ob, [lanes, jnp.full((EDGE_DIM,), i, jnp.int32)], v)

                start_out(k, b)

                @pl.when(k + NBUF < n_c)
                def _():
                    start_in(k + NBUF, b)

    for b in range(NBUF):
        k_last = n_c - 1 - ((n_c - 1 - b) % NBUF)
        wait_out(k_last, b)


def _edge_update(pr, ps, ridx, sidx):
    mesh = plsc.VectorSubcoreMesh(core_axis_name="c", subcore_axis_name="s")
    f = pl.kernel(
        _edge_body,
        out_type=jax.ShapeDtypeStruct((2, N_TCOLS, 8, 128), jnp.float32),
        mesh=mesh,
        scratch_types=[
            pltpu.VMEM((MAX_WCHUNKS * CHUNK,), jnp.int32),
            pltpu.VMEM((MAX_WCHUNKS * CHUNK,), jnp.int32),
        ] + [pltpu.VMEM((CHUNK, EDGE_DIM), jnp.float32)] * 4
          + [pltpu.VMEM((EDGE_DIM, CHUNK + 1), jnp.float32)] * 2
          + [pltpu.VMEM_SHARED((N_NODES, EDGE_DIM), jnp.float32)] * 2 + [
            pltpu.SemaphoreType.DMA,
            pltpu.SemaphoreType.DMA,
            pltpu.SemaphoreType.DMA,
            pltpu.SemaphoreType.DMA,
            pltpu.SemaphoreType.DMA,
        ],
        compiler_params=pltpu.CompilerParams(use_tc_tiling_on_sc=False,
                                             needs_layout_passes=False),
    )
    return f(pr, ps, ridx, sidx)


def kernel(x, senders, receivers, W_base, b_base, W_edge, b_edge):
    pr_c, ps_c = _node_tables(x, W_base, b_base, W_edge, b_edge)
    pr = pr_c.reshape(N_NODES, EDGE_DIM)
    ps = ps_c.reshape(N_NODES, EDGE_DIM)
    out4 = _edge_update(pr, ps, receivers, senders)
    return out4.transpose(1, 3, 0, 2).reshape(N_EDGES, EDGE_DIM)
